# baseline (speedup 1.0000x reference)
import functools

import jax
import jax.numpy as jnp
from jax import lax
from jax.experimental import pallas as pl
from jax.experimental.pallas import tpu as pltpu
from jax.experimental.pallas import tpu_sc as plsc

_D = 1024
_M = 100000
_H = 4
_NC = 2
_NS = 16
_NW = _NC * _NS
_C = 64


def _sc_gather(ids, coeffs_exp, table):
    t = ids.shape[0]
    n_rows = t * _H
    rw = n_rows // _NW
    wph = _NW // _H
    nch = rw // _C
    nvec = rw // 16

    mesh = plsc.VectorSubcoreMesh(core_axis_name="c", subcore_axis_name="s")

    @functools.partial(
        pl.kernel,
        out_type=jax.ShapeDtypeStruct((n_rows, _D), jnp.bfloat16),
        mesh=mesh,
        compiler_params=pltpu.CompilerParams(needs_layout_passes=False),
        scratch_types=[
            pltpu.VMEM((rw,), jnp.int32),
            pltpu.VMEM((16,), jnp.float32),
            pltpu.VMEM((rw,), jnp.int32),
            pltpu.VMEM((_C, _D), jnp.float32),
            pltpu.VMEM((_C, _D), jnp.bfloat16),
            pltpu.SemaphoreType.DMA,
            pltpu.SemaphoreType.DMA,
        ],
    )
    def k(ids_hbm, coef_hbm, table_hbm, out_hbm,
          ids_v, coef_v, idx_v, buf0, pk0, sem0, wsem0):
        wid = lax.axis_index("s") * _NC + lax.axis_index("c")
        base_r = pl.multiple_of(wid * rw, rw)
        base_t = pl.multiple_of(lax.rem(wid, wph) * rw, rw)
        pltpu.sync_copy(ids_hbm.at[pl.ds(base_t, rw)], ids_v)
        pltpu.sync_copy(coef_hbm.at[wid], coef_v)
        coefv = coef_v[...]
        rcp_m = jnp.float32(1.0) / jnp.float32(_M)

        def hash_body(j, carry):
            off = pl.multiple_of(j * 16, 16)
            tok = ids_v[pl.ds(off, 16)]
            p = tok.astype(jnp.float32) * coefv
            tr = (p * rcp_m).astype(jnp.int32).astype(jnp.float32)
            r = (p - tr * jnp.float32(_M)).astype(jnp.int32)
            idx_v[pl.ds(off, 16)] = jnp.minimum(jnp.maximum(r, 0), _M - 1)
            return carry

        lax.fori_loop(0, nvec, hash_body, 0)

        two_iota = 2 * lax.iota(jnp.int32, 16)

        def pack_chunk(buf, pk):
            @plsc.parallel_loop(0, _C, 1)
            def prow(r):
                row = jnp.full((16,), 0, jnp.int32) + r
                for gi in range(_D // 32):
                    col = gi * 32 + two_iota
                    ev = plsc.load_gather(buf, [row, col])
                    od = plsc.load_gather(buf, [row, col + 1])
                    pk[r, pl.ds(gi * 32, 32)] = plsc.pack(
                        ev, od, format=plsc.PackFormat.INTERLEAVED)

        def start(c, buf, sem):
            off = pl.multiple_of(c * _C, _C)
            pltpu.async_copy(table_hbm.at[idx_v.at[pl.ds(off, _C)]], buf, sem)

        def out_slice(c):
            off = pl.multiple_of(c * _C, _C)
            return out_hbm.at[pl.ds(base_r + off, _C)]

        start(0, buf0, sem0)

        def gather_body(c, carry):
            off = pl.multiple_of(c * _C, _C)
            pltpu.make_async_copy(
                table_hbm.at[idx_v.at[pl.ds(off, _C)]], buf0, sem0).wait()

            @pl.when(c > 0)
            def _():
                pltpu.make_async_copy(pk0, out_slice(c - 1), wsem0).wait()

            pack_chunk(buf0, pk0)

            @pl.when(c + 1 < nch)
            def _():
                start(c + 1, buf0, sem0)

            pltpu.async_copy(pk0, out_slice(c), wsem0)
            return carry

        lax.fori_loop(0, nch, gather_body, 0)
        pltpu.make_async_copy(pk0, out_slice(nch - 1), wsem0).wait()

    return k(ids, coeffs_exp, table)


def _tc_dense(g3, w, b2, gate, hidden, t_full, blk0, prev=None):
    ts = g3.shape[1]
    bt = 256

    def body(*refs):
        if prev is None:
            x_ref, w_ref, b_ref, g_ref, h_ref, o_ref = refs
        else:
            x_ref, w_ref, b_ref, g_ref, h_ref, _, o_ref = refs
        acc = lax.dot_general(
            x_ref[0], w_ref[:, : _D], (((1,), (1,)), ((), ())),
            preferred_element_type=jnp.float32)
        for h in range(1, _H):
            acc += lax.dot_general(
                x_ref[h], w_ref[:, h * _D:(h + 1) * _D],
                (((1,), (1,)), ((), ())),
                preferred_element_type=jnp.float32)
        g = g_ref[0]
        o_ref[...] = (1.0 - g) * h_ref[...] + g * (acc + b_ref[...])

    in_specs = [
        pl.BlockSpec((_H, bt, _D), lambda i: (0, i, 0)),
        pl.BlockSpec((_D, _H * _D), lambda i: (0, 0)),
        pl.BlockSpec((1, _D), lambda i: (0, 0)),
        pl.BlockSpec(memory_space=pltpu.SMEM),
        pl.BlockSpec((bt, _D), lambda i: (i + blk0, 0)),
    ]
    args = [g3, w, b2, gate, hidden]
    kwargs = {}
    if prev is not None:
        in_specs.append(pl.BlockSpec(memory_space=pl.ANY))
        args.append(prev)
        kwargs["input_output_aliases"] = {5: 0}
    return pl.pallas_call(
        body,
        grid=(ts // bt,),
        in_specs=in_specs,
        out_specs=pl.BlockSpec((bt, _D), lambda i: (i + blk0, 0)),
        out_shape=jax.ShapeDtypeStruct((t_full, _D), jnp.float32),
        **kwargs,
    )(*args)


_NSLICE = 2


def kernel(hidden_states, input_ids, memory_table, hash_coeffs, W, b, gate):
    batch, seq, d = hidden_states.shape
    t = batch * seq
    ts = t // _NSLICE
    coeffs_exp = jnp.broadcast_to(
        jnp.repeat(hash_coeffs, _NW // _H)[:, None], (_NW, 16))
    ids_flat = input_ids.reshape(t)
    hidden2 = hidden_states.reshape(t, d)
    b2 = b.reshape(1, d)
    w16 = W.astype(jnp.bfloat16)
    gs = [_sc_gather(ids_flat[s * ts:(s + 1) * ts], coeffs_exp, memory_table)
          for s in range(_NSLICE)]
    out = None
    for s in range(_NSLICE):
        out = _tc_dense(gs[s].reshape(_H, ts, d), w16, b2, gate, hidden2,
                        t, s * (ts // 256), prev=out)
    return out.reshape(batch, seq, d)


# speedup vs baseline: 1.2603x; 1.2603x over previous
import functools

import jax
import jax.numpy as jnp
from jax import lax
from jax.experimental import pallas as pl
from jax.experimental.pallas import tpu as pltpu
from jax.experimental.pallas import tpu_sc as plsc

_D = 1024
_M = 100000
_H = 4
_NC = 2
_NS = 16
_NW = _NC * _NS
_C = 32


def _sc_gather(ids, coeffs_exp, table):
    t = ids.shape[0]
    n_rows = t * _H
    rw = n_rows // _NW
    wph = _NW // _H
    nch = rw // _C
    nvec = rw // 16

    mesh = plsc.VectorSubcoreMesh(core_axis_name="c", subcore_axis_name="s")

    @functools.partial(
        pl.kernel,
        out_type=jax.ShapeDtypeStruct((n_rows, _D), jnp.bfloat16),
        mesh=mesh,
        compiler_params=pltpu.CompilerParams(needs_layout_passes=False),
        scratch_types=[
            pltpu.VMEM((rw,), jnp.int32),
            pltpu.VMEM((16,), jnp.float32),
            pltpu.VMEM((rw,), jnp.int32),
            pltpu.VMEM((_C, _D), jnp.float32),
            pltpu.VMEM((_C, _D), jnp.float32),
            pltpu.VMEM((_C, _D), jnp.bfloat16),
            pltpu.VMEM((_C, _D), jnp.bfloat16),
            pltpu.SemaphoreType.DMA,
            pltpu.SemaphoreType.DMA,
            pltpu.SemaphoreType.DMA,
            pltpu.SemaphoreType.DMA,
        ],
    )
    def k(ids_hbm, coef_hbm, table_hbm, out_hbm,
          ids_v, coef_v, idx_v, buf0, buf1, pk0, pk1,
          sem0, sem1, wsem0, wsem1):
        wid = lax.axis_index("s") * _NC + lax.axis_index("c")
        base_r = pl.multiple_of(wid * rw, rw)
        base_t = pl.multiple_of(lax.rem(wid, wph) * rw, rw)
        pltpu.sync_copy(ids_hbm.at[pl.ds(base_t, rw)], ids_v)
        pltpu.sync_copy(coef_hbm.at[wid], coef_v)
        coefv = coef_v[...]
        rcp_m = jnp.float32(1.0) / jnp.float32(_M)

        def hash_body(j, carry):
            off = pl.multiple_of(j * 16, 16)
            tok = ids_v[pl.ds(off, 16)]
            p = tok.astype(jnp.float32) * coefv
            tr = (p * rcp_m).astype(jnp.int32).astype(jnp.float32)
            r = (p - tr * jnp.float32(_M)).astype(jnp.int32)
            idx_v[pl.ds(off, 16)] = jnp.minimum(jnp.maximum(r, 0), _M - 1)
            return carry

        lax.fori_loop(0, nvec, hash_body, 0)

        two_iota = 2 * lax.iota(jnp.int32, 16)

        def pack_chunk(buf, pk):
            @plsc.parallel_loop(0, _C, 1)
            def prow(r):
                row = jnp.full((16,), 0, jnp.int32) + r
                for gi in range(_D // 32):
                    col = gi * 32 + two_iota
                    ev = plsc.load_gather(buf, [row, col])
                    od = plsc.load_gather(buf, [row, col + 1])
                    pk[r, pl.ds(gi * 32, 32)] = plsc.pack(
                        ev, od, format=plsc.PackFormat.INTERLEAVED)

        def start(c, buf, sem):
            off = pl.multiple_of(c * _C, _C)
            pltpu.async_copy(table_hbm.at[idx_v.at[pl.ds(off, _C)]], buf, sem)

        def out_slice(c):
            off = pl.multiple_of(c * _C, _C)
            return out_hbm.at[pl.ds(base_r + off, _C)]

        def finish(g, c, buf, pk, sem, wsem):
            off = pl.multiple_of(c * _C, _C)
            pltpu.make_async_copy(
                table_hbm.at[idx_v.at[pl.ds(off, _C)]], buf, sem).wait()

            @pl.when(g > 0)
            def _():
                pltpu.make_async_copy(pk, out_slice(c - 2), wsem).wait()

            pack_chunk(buf, pk)
            pltpu.async_copy(pk, out_slice(c), wsem)

        start(0, buf0, sem0)

        def gather_body(g, carry):
            c0 = 2 * g
            start(c0 + 1, buf1, sem1)
            finish(g, c0, buf0, pk0, sem0, wsem0)

            @pl.when(c0 + 2 < nch)
            def _():
                start(c0 + 2, buf0, sem0)

            finish(g, c0 + 1, buf1, pk1, sem1, wsem1)
            return carry

        lax.fori_loop(0, nch // 2, gather_body, 0)
        pltpu.make_async_copy(pk0, out_slice(nch - 2), wsem0).wait()
        pltpu.make_async_copy(pk1, out_slice(nch - 1), wsem1).wait()

    return k(ids, coeffs_exp, table)


def _tc_dense(g3, w, b2, gate, hidden, t_full, blk0, prev=None):
    ts = g3.shape[1]
    bt = 256

    def body(*refs):
        if prev is None:
            x_ref, w_ref, b_ref, g_ref, h_ref, o_ref = refs
        else:
            x_ref, w_ref, b_ref, g_ref, h_ref, _, o_ref = refs
        acc = lax.dot_general(
            x_ref[0], w_ref[:, : _D], (((1,), (1,)), ((), ())),
            preferred_element_type=jnp.float32)
        for h in range(1, _H):
            acc += lax.dot_general(
                x_ref[h], w_ref[:, h * _D:(h + 1) * _D],
                (((1,), (1,)), ((), ())),
                preferred_element_type=jnp.float32)
        g = g_ref[0]
        o_ref[...] = (1.0 - g) * h_ref[...] + g * (acc + b_ref[...])

    in_specs = [
        pl.BlockSpec((_H, bt, _D), lambda i: (0, i, 0)),
        pl.BlockSpec((_D, _H * _D), lambda i: (0, 0)),
        pl.BlockSpec((1, _D), lambda i: (0, 0)),
        pl.BlockSpec(memory_space=pltpu.SMEM),
        pl.BlockSpec((bt, _D), lambda i: (i + blk0, 0)),
    ]
    args = [g3, w, b2, gate, hidden]
    kwargs = {}
    if prev is not None:
        in_specs.append(pl.BlockSpec(memory_space=pl.ANY))
        args.append(prev)
        kwargs["input_output_aliases"] = {5: 0}
    return pl.pallas_call(
        body,
        grid=(ts // bt,),
        in_specs=in_specs,
        out_specs=pl.BlockSpec((bt, _D), lambda i: (i + blk0, 0)),
        out_shape=jax.ShapeDtypeStruct((t_full, _D), jnp.float32),
        **kwargs,
    )(*args)


_NSLICE = 2


def kernel(hidden_states, input_ids, memory_table, hash_coeffs, W, b, gate):
    batch, seq, d = hidden_states.shape
    t = batch * seq
    ts = t // _NSLICE
    coeffs_exp = jnp.broadcast_to(
        jnp.repeat(hash_coeffs, _NW // _H)[:, None], (_NW, 16))
    ids_flat = input_ids.reshape(t)
    hidden2 = hidden_states.reshape(t, d)
    b2 = b.reshape(1, d)
    w16 = W.astype(jnp.bfloat16)
    gs = [_sc_gather(ids_flat[s * ts:(s + 1) * ts], coeffs_exp, memory_table)
          for s in range(_NSLICE)]
    out = None
    for s in range(_NSLICE):
        out = _tc_dense(gs[s].reshape(_H, ts, d), w16, b2, gate, hidden2,
                        t, s * (ts // 256), prev=out)
    return out.reshape(batch, seq, d)


# speedup vs baseline: 1.2790x; 1.0148x over previous
import functools

import jax
import jax.numpy as jnp
from jax import lax
from jax.experimental import pallas as pl
from jax.experimental.pallas import tpu as pltpu
from jax.experimental.pallas import tpu_sc as plsc

_D = 1024
_M = 100000
_H = 4
_NC = 2
_NS = 16
_NW = _NC * _NS
_C = 32


def _sc_gather(ids, coeffs_exp, table):
    t = ids.shape[0]
    n_rows = t * _H
    rw = n_rows // _NW
    wph = _NW // _H
    nch = rw // _C
    nvec = rw // 16

    mesh = plsc.VectorSubcoreMesh(core_axis_name="c", subcore_axis_name="s")

    @functools.partial(
        pl.kernel,
        out_type=jax.ShapeDtypeStruct((n_rows, _D), jnp.bfloat16),
        mesh=mesh,
        compiler_params=pltpu.CompilerParams(needs_layout_passes=False),
        scratch_types=[
            pltpu.VMEM((rw,), jnp.int32),
            pltpu.VMEM((16,), jnp.float32),
            pltpu.VMEM((rw,), jnp.int32),
            pltpu.VMEM((_C, _D), jnp.float32),
            pltpu.VMEM((_C, _D), jnp.float32),
            pltpu.VMEM((_C, _D), jnp.bfloat16),
            pltpu.VMEM((_C, _D), jnp.bfloat16),
            pltpu.SemaphoreType.DMA,
            pltpu.SemaphoreType.DMA,
            pltpu.SemaphoreType.DMA,
            pltpu.SemaphoreType.DMA,
        ],
    )
    def k(ids_hbm, coef_hbm, table_hbm, out_hbm,
          ids_v, coef_v, idx_v, buf0, buf1, pk0, pk1,
          sem0, sem1, wsem0, wsem1):
        wid = lax.axis_index("s") * _NC + lax.axis_index("c")
        base_r = pl.multiple_of(wid * rw, rw)
        base_t = pl.multiple_of(lax.rem(wid, wph) * rw, rw)
        pltpu.sync_copy(ids_hbm.at[pl.ds(base_t, rw)], ids_v)
        pltpu.sync_copy(coef_hbm.at[wid], coef_v)
        coefv = coef_v[...]
        rcp_m = jnp.float32(1.0) / jnp.float32(_M)

        def hash_body(j, carry):
            off = pl.multiple_of(j * 16, 16)
            tok = ids_v[pl.ds(off, 16)]
            p = tok.astype(jnp.float32) * coefv
            tr = (p * rcp_m).astype(jnp.int32).astype(jnp.float32)
            r = (p - tr * jnp.float32(_M)).astype(jnp.int32)
            idx_v[pl.ds(off, 16)] = jnp.minimum(jnp.maximum(r, 0), _M - 1)
            return carry

        lax.fori_loop(0, nvec, hash_body, 0)

        two_iota = 2 * lax.iota(jnp.int32, 16)

        def pack_chunk(buf, pk):
            @plsc.parallel_loop(0, _C, 1)
            def prow(r):
                row = jnp.full((16,), 0, jnp.int32) + r
                for gi in range(_D // 32):
                    col = gi * 32 + two_iota
                    ev = plsc.load_gather(buf, [row, col])
                    od = plsc.load_gather(buf, [row, col + 1])
                    pk[r, pl.ds(gi * 32, 32)] = plsc.pack(
                        ev, od, format=plsc.PackFormat.INTERLEAVED)

        def start(c, buf, sem):
            off = pl.multiple_of(c * _C, _C)
            pltpu.async_copy(table_hbm.at[idx_v.at[pl.ds(off, _C)]], buf, sem)

        def out_slice(c):
            off = pl.multiple_of(c * _C, _C)
            return out_hbm.at[pl.ds(base_r + off, _C)]

        def finish(g, c, buf, pk, sem, wsem):
            off = pl.multiple_of(c * _C, _C)
            pltpu.make_async_copy(
                table_hbm.at[idx_v.at[pl.ds(off, _C)]], buf, sem).wait()

            @pl.when(g > 0)
            def _():
                pltpu.make_async_copy(pk, out_slice(c - 2), wsem).wait()

            pack_chunk(buf, pk)
            pltpu.async_copy(pk, out_slice(c), wsem)

        start(0, buf0, sem0)

        def gather_body(g, carry):
            c0 = 2 * g
            start(c0 + 1, buf1, sem1)
            finish(g, c0, buf0, pk0, sem0, wsem0)

            @pl.when(c0 + 2 < nch)
            def _():
                start(c0 + 2, buf0, sem0)

            finish(g, c0 + 1, buf1, pk1, sem1, wsem1)
            return carry

        lax.fori_loop(0, nch // 2, gather_body, 0)
        pltpu.make_async_copy(pk0, out_slice(nch - 2), wsem0).wait()
        pltpu.make_async_copy(pk1, out_slice(nch - 1), wsem1).wait()

    return k(ids, coeffs_exp, table)


def _tc_dense(g3, w, b2, gate, hidden, t_full, blk0, prev=None):
    ts = g3.shape[1]
    bt = 256

    def body(*refs):
        if prev is None:
            x_ref, w_ref, b_ref, g_ref, h_ref, o_ref = refs
        else:
            x_ref, w_ref, b_ref, g_ref, h_ref, _, o_ref = refs
        acc = lax.dot_general(
            x_ref[0], w_ref[:, : _D].astype(jnp.bfloat16),
            (((1,), (1,)), ((), ())),
            preferred_element_type=jnp.float32)
        for h in range(1, _H):
            acc += lax.dot_general(
                x_ref[h], w_ref[:, h * _D:(h + 1) * _D].astype(jnp.bfloat16),
                (((1,), (1,)), ((), ())),
                preferred_element_type=jnp.float32)
        g = g_ref[0]
        o_ref[...] = (1.0 - g) * h_ref[...] + g * (acc + b_ref[...])

    in_specs = [
        pl.BlockSpec((_H, bt, _D), lambda i: (0, i, 0)),
        pl.BlockSpec((_D, _H * _D), lambda i: (0, 0)),
        pl.BlockSpec((1, _D), lambda i: (0, 0)),
        pl.BlockSpec(memory_space=pltpu.SMEM),
        pl.BlockSpec((bt, _D), lambda i: (i + blk0, 0)),
    ]
    args = [g3, w, b2, gate, hidden]
    kwargs = {}
    if prev is not None:
        in_specs.append(pl.BlockSpec(memory_space=pl.ANY))
        args.append(prev)
        kwargs["input_output_aliases"] = {5: 0}
    return pl.pallas_call(
        body,
        grid=(ts // bt,),
        in_specs=in_specs,
        out_specs=pl.BlockSpec((bt, _D), lambda i: (i + blk0, 0)),
        out_shape=jax.ShapeDtypeStruct((t_full, _D), jnp.float32),
        **kwargs,
    )(*args)


_NSLICE = 2


def kernel(hidden_states, input_ids, memory_table, hash_coeffs, W, b, gate):
    batch, seq, d = hidden_states.shape
    t = batch * seq
    ts = t // _NSLICE
    coeffs_exp = jnp.broadcast_to(
        jnp.repeat(hash_coeffs, _NW // _H)[:, None], (_NW, 16))
    ids_flat = input_ids.reshape(t)
    hidden2 = hidden_states.reshape(t, d)
    b2 = b.reshape(1, d)
    gs = [_sc_gather(ids_flat[s * ts:(s + 1) * ts], coeffs_exp, memory_table)
          for s in range(_NSLICE)]
    out = None
    for s in range(_NSLICE):
        out = _tc_dense(gs[s].reshape(_H, ts, d), W, b2, gate, hidden2,
                        t, s * (ts // 256), prev=out)
    return out.reshape(batch, seq, d)


# speedup vs baseline: 1.2820x; 1.0023x over previous
import functools

import jax
import jax.numpy as jnp
from jax import lax
from jax.experimental import pallas as pl
from jax.experimental.pallas import tpu as pltpu
from jax.experimental.pallas import tpu_sc as plsc

_D = 1024
_M = 100000
_H = 4
_NC = 2
_NS = 16
_NW = _NC * _NS
_C = 32


def _sc_gather(ids, coeffs_exp, table):
    t = ids.shape[0]
    n_rows = t * _H
    rw = n_rows // _NW
    wph = _NW // _H
    nch = rw // _C
    nvec = rw // 16

    mesh = plsc.VectorSubcoreMesh(core_axis_name="c", subcore_axis_name="s")

    @functools.partial(
        pl.kernel,
        out_type=jax.ShapeDtypeStruct((n_rows, _D), jnp.bfloat16),
        mesh=mesh,
        compiler_params=pltpu.CompilerParams(needs_layout_passes=False),
        scratch_types=[
            pltpu.VMEM((rw,), jnp.int32),
            pltpu.VMEM((16,), jnp.float32),
            pltpu.VMEM((rw,), jnp.int32),
            pltpu.VMEM((_C, _D), jnp.float32),
            pltpu.VMEM((_C, _D), jnp.float32),
            pltpu.VMEM((_C, _D), jnp.bfloat16),
            pltpu.VMEM((_C, _D), jnp.bfloat16),
            pltpu.SemaphoreType.DMA,
            pltpu.SemaphoreType.DMA,
            pltpu.SemaphoreType.DMA,
            pltpu.SemaphoreType.DMA,
        ],
    )
    def k(ids_hbm, coef_hbm, table_hbm, out_hbm,
          ids_v, coef_v, idx_v, buf0, buf1, pk0, pk1,
          sem0, sem1, wsem0, wsem1):
        wid = lax.axis_index("s") * _NC + lax.axis_index("c")
        base_r = pl.multiple_of(wid * rw, rw)
        base_t = pl.multiple_of(lax.rem(wid, wph) * rw, rw)
        pltpu.sync_copy(ids_hbm.at[pl.ds(base_t, rw)], ids_v)
        pltpu.sync_copy(coef_hbm.at[wid], coef_v)
        coefv = coef_v[...]
        rcp_m = jnp.float32(1.0) / jnp.float32(_M)

        def hash_body(j, carry):
            off = pl.multiple_of(j * 16, 16)
            tok = ids_v[pl.ds(off, 16)]
            p = tok.astype(jnp.float32) * coefv
            tr = (p * rcp_m).astype(jnp.int32).astype(jnp.float32)
            r = (p - tr * jnp.float32(_M)).astype(jnp.int32)
            idx_v[pl.ds(off, 16)] = jnp.minimum(jnp.maximum(r, 0), _M - 1)
            return carry

        lax.fori_loop(0, nvec, hash_body, 0)

        two_iota = 2 * lax.iota(jnp.int32, 16)

        def pack_chunk(buf, pk):
            @plsc.parallel_loop(0, _C, 1)
            def prow(r):
                row = jnp.full((16,), 0, jnp.int32) + r
                for gi in range(_D // 32):
                    col = gi * 32 + two_iota
                    ev = plsc.load_gather(buf, [row, col])
                    od = plsc.load_gather(buf, [row, col + 1])
                    pk[r, pl.ds(gi * 32, 32)] = plsc.pack(
                        ev, od, format=plsc.PackFormat.INTERLEAVED)

        def start(c, buf, sem):
            off = pl.multiple_of(c * _C, _C)
            pltpu.async_copy(table_hbm.at[idx_v.at[pl.ds(off, _C)]], buf, sem)

        def out_slice(c):
            off = pl.multiple_of(c * _C, _C)
            return out_hbm.at[pl.ds(base_r + off, _C)]

        def finish(g, c, buf, pk, sem, wsem):
            off = pl.multiple_of(c * _C, _C)
            pltpu.make_async_copy(
                table_hbm.at[idx_v.at[pl.ds(off, _C)]], buf, sem).wait()

            @pl.when(g > 0)
            def _():
                pltpu.make_async_copy(pk, out_slice(c - 2), wsem).wait()

            pack_chunk(buf, pk)
            pltpu.async_copy(pk, out_slice(c), wsem)

        start(0, buf0, sem0)

        def gather_body(g, carry):
            c0 = 2 * g
            start(c0 + 1, buf1, sem1)
            finish(g, c0, buf0, pk0, sem0, wsem0)

            @pl.when(c0 + 2 < nch)
            def _():
                start(c0 + 2, buf0, sem0)

            finish(g, c0 + 1, buf1, pk1, sem1, wsem1)
            return carry

        lax.fori_loop(0, nch // 2, gather_body, 0)
        pltpu.make_async_copy(pk0, out_slice(nch - 2), wsem0).wait()
        pltpu.make_async_copy(pk1, out_slice(nch - 1), wsem1).wait()

    return k(ids, coeffs_exp, table)


def _tc_dense(g3, w, b2, gate, hidden, t_full, blk0, prev=None):
    ts = g3.shape[1]
    bt = 512

    def body(*refs):
        if prev is None:
            x_ref, w_ref, b_ref, g_ref, h_ref, o_ref = refs
        else:
            x_ref, w_ref, b_ref, g_ref, h_ref, _, o_ref = refs
        acc = lax.dot_general(
            x_ref[0], w_ref[:, : _D].astype(jnp.bfloat16),
            (((1,), (1,)), ((), ())),
            preferred_element_type=jnp.float32)
        for h in range(1, _H):
            acc += lax.dot_general(
                x_ref[h], w_ref[:, h * _D:(h + 1) * _D].astype(jnp.bfloat16),
                (((1,), (1,)), ((), ())),
                preferred_element_type=jnp.float32)
        g = g_ref[0]
        o_ref[...] = (1.0 - g) * h_ref[...] + g * (acc + b_ref[...])

    in_specs = [
        pl.BlockSpec((_H, bt, _D), lambda i: (0, i, 0)),
        pl.BlockSpec((_D, _H * _D), lambda i: (0, 0)),
        pl.BlockSpec((1, _D), lambda i: (0, 0)),
        pl.BlockSpec(memory_space=pltpu.SMEM),
        pl.BlockSpec((bt, _D), lambda i: (i + blk0, 0)),
    ]
    args = [g3, w, b2, gate, hidden]
    kwargs = {}
    if prev is not None:
        in_specs.append(pl.BlockSpec(memory_space=pl.ANY))
        args.append(prev)
        kwargs["input_output_aliases"] = {5: 0}
    return pl.pallas_call(
        body,
        grid=(ts // bt,),
        in_specs=in_specs,
        out_specs=pl.BlockSpec((bt, _D), lambda i: (i + blk0, 0)),
        out_shape=jax.ShapeDtypeStruct((t_full, _D), jnp.float32),
        **kwargs,
    )(*args)


_NSLICE = 2


def kernel(hidden_states, input_ids, memory_table, hash_coeffs, W, b, gate):
    batch, seq, d = hidden_states.shape
    t = batch * seq
    ts = t // _NSLICE
    coeffs_exp = jnp.broadcast_to(
        jnp.repeat(hash_coeffs, _NW // _H)[:, None], (_NW, 16))
    ids_flat = input_ids.reshape(t)
    hidden2 = hidden_states.reshape(t, d)
    b2 = b.reshape(1, d)
    gs = [_sc_gather(ids_flat[s * ts:(s + 1) * ts], coeffs_exp, memory_table)
          for s in range(_NSLICE)]
    out = None
    for s in range(_NSLICE):
        out = _tc_dense(gs[s].reshape(_H, ts, d), W, b2, gate, hidden2,
                        t, s * (ts // 512), prev=out)
    return out.reshape(batch, seq, d)
